# Initial kernel scaffold; baseline (speedup 1.0000x reference)
import functools

import jax
import jax.numpy as jnp
from jax import lax
from jax.experimental import pallas as pl
from jax.experimental.pallas import tpu as pltpu
from jax.experimental.pallas import tpu_sc as plsc

N = 10000
E = 320000
D = 128

NC = 2
NS = 16
NW = NC * NS

N_PAD = 10240
ROWS_PT = N_PAD // NS
CH = 128
NCH = 80
GRP = 4
E_PAD = NW * NCH * CH

BLK = 1280
GRID = N_PAD // BLK

_mesh = plsc.VectorSubcoreMesh(core_axis_name="c", subcore_axis_name="s")


def _wid():
    return lax.axis_index("s") * NC + lax.axis_index("c")


@functools.partial(
    pl.kernel,
    out_type=jax.ShapeDtypeStruct((NC, N_PAD, 16), jnp.float32),
    mesh=_mesh,
    scratch_types=[
        pltpu.VMEM((NCH, CH), jnp.int32),
        pltpu.VMEM((CH, 16), jnp.float32),
        pltpu.VMEM_SHARED((N_PAD, 16), jnp.float32),
    ],
)
def _deg_kernel(dst_hbm, ones_hbm, zeros_hbm, out_hbm, dst_v, ones_v, acc):
    c = lax.axis_index("c")
    t = lax.axis_index("s")
    w = _wid()
    sl = pl.ds(t * ROWS_PT, ROWS_PT)

    pltpu.sync_copy(dst_hbm.at[pl.ds(w * NCH, NCH)], dst_v)
    pltpu.sync_copy(ones_hbm.at[pl.ds(0, CH)], ones_v)

    @pl.when(c == 0)
    def _():
        pltpu.sync_copy(ones_hbm.at[sl], acc.at[sl])

    @pl.when(c != 0)
    def _():
        pltpu.sync_copy(zeros_hbm.at[sl], acc.at[sl])

    plsc.subcore_barrier()

    def body(i, carry):
        for k in range(GRP):
            pltpu.sync_copy(ones_v, acc.at[dst_v.at[i * GRP + k]], add=True)
        return carry

    lax.fori_loop(0, NCH // GRP, body, 0)

    plsc.subcore_barrier()
    pltpu.sync_copy(acc.at[sl], out_hbm.at[c, sl])


@functools.partial(
    pl.kernel,
    out_type=jax.ShapeDtypeStruct((NC, N_PAD, D), jnp.float32),
    mesh=_mesh,
    scratch_types=[
        pltpu.VMEM((NCH, CH), jnp.int32),
        pltpu.VMEM((NCH, CH), jnp.int32),
        pltpu.VMEM((GRP, CH, D), jnp.float32),
        pltpu.VMEM_SHARED((N_PAD, D), jnp.float32),
        pltpu.SemaphoreType.DMA,
    ],
)
def _edge_kernel(y_hbm, src_hbm, dst_hbm, zeros_hbm, out_hbm,
                 src_v, dst_v, rows_v, acc, sem):
    c = lax.axis_index("c")
    t = lax.axis_index("s")
    w = _wid()
    sl = pl.ds(t * ROWS_PT, ROWS_PT)

    pltpu.sync_copy(src_hbm.at[pl.ds(w * NCH, NCH)], src_v)
    pltpu.sync_copy(dst_hbm.at[pl.ds(w * NCH, NCH)], dst_v)

    @pl.when(c == 0)
    def _():
        pltpu.sync_copy(y_hbm.at[sl], acc.at[sl])

    @pl.when(c != 0)
    def _():
        pltpu.sync_copy(zeros_hbm.at[sl], acc.at[sl])

    plsc.subcore_barrier()

    def body(i, carry):
        base = i * GRP
        descs = [
            pltpu.async_copy(y_hbm.at[src_v.at[base + k]], rows_v.at[k], sem)
            for k in range(GRP)
        ]
        for d_ in descs:
            d_.wait()
        for k in range(GRP):
            pltpu.sync_copy(rows_v.at[k], acc.at[dst_v.at[base + k]], add=True)
        return carry

    lax.fori_loop(0, NCH // GRP, body, 0)

    plsc.subcore_barrier()
    pltpu.sync_copy(acc.at[sl], out_hbm.at[c, sl])


def _tc_b_body(x_ref, degp_ref, w_ref, y_ref, dis_ref):
    deg = degp_ref[0, :, 0:1] + degp_ref[1, :, 0:1]
    disb = jnp.broadcast_to(lax.rsqrt(deg), (BLK, D))
    xw = jnp.dot(x_ref[...], w_ref[...], preferred_element_type=jnp.float32)
    y_ref[...] = xw * disb
    dis_ref[...] = disb


def _tc_d_body(p_ref, dis_ref, w_ref, b_ref, y2_ref):
    dis = dis_ref[...]
    h = jnp.maximum((p_ref[0] + p_ref[1]) * dis + b_ref[...], 0.0)
    y2_ref[...] = jnp.dot(h, w_ref[...], preferred_element_type=jnp.float32) * dis


def _tc_f_body(p_ref, dis_ref, b_ref, o_ref):
    o_ref[...] = (p_ref[0] + p_ref[1]) * dis_ref[...] + b_ref[...]


_row_spec = pl.BlockSpec((BLK, D), lambda i: (i, 0))
_part_spec = pl.BlockSpec((NC, BLK, D), lambda i: (0, i, 0))
_full_spec = pl.BlockSpec((D, D), lambda i: (0, 0))
_bias_spec = pl.BlockSpec((1, D), lambda i: (0, 0))

_tc_b = pl.pallas_call(
    _tc_b_body,
    grid=(GRID,),
    in_specs=[_row_spec, pl.BlockSpec((NC, BLK, 16), lambda i: (0, i, 0)),
              _full_spec],
    out_specs=[_row_spec, _row_spec],
    out_shape=[jax.ShapeDtypeStruct((N_PAD, D), jnp.float32),
               jax.ShapeDtypeStruct((N_PAD, D), jnp.float32)],
)

_tc_d = pl.pallas_call(
    _tc_d_body,
    grid=(GRID,),
    in_specs=[_part_spec, _row_spec, _full_spec, _bias_spec],
    out_specs=_row_spec,
    out_shape=jax.ShapeDtypeStruct((N_PAD, D), jnp.float32),
)

_tc_f = pl.pallas_call(
    _tc_f_body,
    grid=(GRID,),
    in_specs=[_part_spec, _row_spec, _bias_spec],
    out_specs=_row_spec,
    out_shape=jax.ShapeDtypeStruct((N_PAD, D), jnp.float32),
)


def kernel(x, edge_index, W1, b1, W2, b2):
    src = edge_index[0].astype(jnp.int32)
    dst = edge_index[1].astype(jnp.int32)

    pad = jnp.full((E_PAD - E,), N, dtype=jnp.int32)
    srcp = jnp.concatenate([src, pad]).reshape(NW * NCH, CH)
    dstp = jnp.concatenate([dst, pad]).reshape(NW * NCH, CH)

    x_pad = jnp.pad(x, ((0, N_PAD - N), (0, 0)))
    zeros_d = jnp.zeros((N_PAD, D), jnp.float32)
    ones_16 = jnp.ones((N_PAD, 16), jnp.float32)
    zeros_16 = jnp.zeros((N_PAD, 16), jnp.float32)

    degp = _deg_kernel(dstp, ones_16, zeros_16)
    y1, dis = _tc_b(x_pad, degp, W1)
    p1 = _edge_kernel(y1, srcp, dstp, zeros_d)
    y2 = _tc_d(p1, dis, W2, b1.reshape(1, D))
    p2 = _edge_kernel(y2, srcp, dstp, zeros_d)
    out = _tc_f(p2, dis, b2.reshape(1, D))
    return out[:N]


# speedup vs baseline: 12.0457x; 12.0457x over previous
import functools

import jax
import jax.numpy as jnp
from jax import lax
from jax.experimental import pallas as pl
from jax.experimental.pallas import tpu as pltpu
from jax.experimental.pallas import tpu_sc as plsc

N = 10000
E = 320000
D = 128
DH = D // 2

NC = 2
NS = 16
NW = NC * NS

N_PAD = 10240
ROWS_PT = N_PAD // NS
CH = 128
NROW = 2560
E_PAD = NROW * CH
NCH_DEG = NROW // NW
NCH_EDGE = NROW // NS
GRP = 4

BLK = 1280
GRID = N_PAD // BLK

_mesh = plsc.VectorSubcoreMesh(core_axis_name="c", subcore_axis_name="s")


@functools.partial(
    pl.kernel,
    out_type=jax.ShapeDtypeStruct((NC, N_PAD, 16), jnp.float32),
    mesh=_mesh,
    scratch_types=[
        pltpu.VMEM((NCH_DEG, CH), jnp.int32),
        pltpu.VMEM((CH, 16), jnp.float32),
        pltpu.VMEM_SHARED((N_PAD, 16), jnp.float32),
    ],
)
def _deg_kernel(dst_hbm, ones_hbm, zeros_hbm, out_hbm, dst_v, ones_v, acc):
    c = lax.axis_index("c")
    t = lax.axis_index("s")
    w = t * NC + c
    sl = pl.ds(t * ROWS_PT, ROWS_PT)

    pltpu.sync_copy(dst_hbm.at[pl.ds(w * NCH_DEG, NCH_DEG)], dst_v)
    pltpu.sync_copy(ones_hbm.at[pl.ds(0, CH)], ones_v)

    @pl.when(c == 0)
    def _():
        pltpu.sync_copy(ones_hbm.at[sl], acc.at[sl])

    @pl.when(c != 0)
    def _():
        pltpu.sync_copy(zeros_hbm.at[sl], acc.at[sl])

    plsc.subcore_barrier()

    def body(i, carry):
        for k in range(GRP):
            pltpu.sync_copy(ones_v, acc.at[dst_v.at[i * GRP + k]], add=True)
        return carry

    lax.fori_loop(0, NCH_DEG // GRP, body, 0)

    plsc.subcore_barrier()
    pltpu.sync_copy(acc.at[sl], out_hbm.at[c, sl])


@functools.partial(
    pl.kernel,
    out_type=jax.ShapeDtypeStruct((NC, N_PAD, DH), jnp.float32),
    mesh=_mesh,
    scratch_types=[
        pltpu.VMEM((NCH_EDGE, CH), jnp.int32),
        pltpu.VMEM((NCH_EDGE, CH), jnp.int32),
        pltpu.VMEM((GRP, CH, DH), jnp.float32),
        pltpu.VMEM_SHARED((N_PAD, DH), jnp.float32),
        pltpu.SemaphoreType.DMA,
    ],
    compiler_params=pltpu.CompilerParams(use_tc_tiling_on_sc=False),
)
def _edge_kernel(y_hbm, src_hbm, dst_hbm, out_hbm, src_v, dst_v, rows_v, acc,
                 sem):
    c = lax.axis_index("c")
    t = lax.axis_index("s")
    sl = pl.ds(t * ROWS_PT, ROWS_PT)
    y_c = y_hbm.at[c]

    pltpu.sync_copy(src_hbm.at[pl.ds(t * NCH_EDGE, NCH_EDGE)], src_v)
    pltpu.sync_copy(dst_hbm.at[pl.ds(t * NCH_EDGE, NCH_EDGE)], dst_v)

    pltpu.sync_copy(y_c.at[sl], acc.at[sl])
    plsc.subcore_barrier()

    def body(i, carry):
        base = i * GRP
        descs = [
            pltpu.async_copy(y_c.at[src_v.at[base + k]], rows_v.at[k], sem)
            for k in range(GRP)
        ]
        for d_ in descs:
            d_.wait()
        for k in range(GRP):
            pltpu.sync_copy(rows_v.at[k], acc.at[dst_v.at[base + k]], add=True)
        return carry

    lax.fori_loop(0, NCH_EDGE // GRP, body, 0)

    plsc.subcore_barrier()
    pltpu.sync_copy(acc.at[sl], out_hbm.at[c, sl])


def _tc_b_body(x_ref, degp_ref, w_ref, y_ref, dis_ref):
    deg = degp_ref[0, :, 0:1] + degp_ref[1, :, 0:1]
    disb = jnp.broadcast_to(lax.rsqrt(deg), (BLK, D))
    xw = jnp.dot(x_ref[...], w_ref[...], preferred_element_type=jnp.float32)
    y = xw * disb
    y_ref[0] = y[:, :DH]
    y_ref[1] = y[:, DH:]
    dis_ref[...] = disb


def _tc_d_body(p_ref, dis_ref, w_ref, b_ref, y2_ref):
    dis = dis_ref[...]
    pre = jnp.concatenate([p_ref[0], p_ref[1]], axis=1)
    h = jnp.maximum(pre * dis + b_ref[...], 0.0)
    y2 = jnp.dot(h, w_ref[...], preferred_element_type=jnp.float32) * dis
    y2_ref[0] = y2[:, :DH]
    y2_ref[1] = y2[:, DH:]


def _tc_f_body(p_ref, dis_ref, b_ref, o_ref):
    pre = jnp.concatenate([p_ref[0], p_ref[1]], axis=1)
    o_ref[...] = pre * dis_ref[...] + b_ref[...]


_row_spec = pl.BlockSpec((BLK, D), lambda i: (i, 0))
_half_spec = pl.BlockSpec((NC, BLK, DH), lambda i: (0, i, 0))
_full_spec = pl.BlockSpec((D, D), lambda i: (0, 0))
_bias_spec = pl.BlockSpec((1, D), lambda i: (0, 0))
_y3 = jax.ShapeDtypeStruct((NC, N_PAD, DH), jnp.float32)

_tc_b = pl.pallas_call(
    _tc_b_body,
    grid=(GRID,),
    in_specs=[_row_spec, pl.BlockSpec((NC, BLK, 16), lambda i: (0, i, 0)),
              _full_spec],
    out_specs=[_half_spec, _row_spec],
    out_shape=[_y3, jax.ShapeDtypeStruct((N_PAD, D), jnp.float32)],
)

_tc_d = pl.pallas_call(
    _tc_d_body,
    grid=(GRID,),
    in_specs=[_half_spec, _row_spec, _full_spec, _bias_spec],
    out_specs=_half_spec,
    out_shape=_y3,
)

_tc_f = pl.pallas_call(
    _tc_f_body,
    grid=(GRID,),
    in_specs=[_half_spec, _row_spec, _bias_spec],
    out_specs=_row_spec,
    out_shape=jax.ShapeDtypeStruct((N_PAD, D), jnp.float32),
)


def kernel(x, edge_index, W1, b1, W2, b2):
    src = edge_index[0].astype(jnp.int32)
    dst = edge_index[1].astype(jnp.int32)

    pad = jnp.full((E_PAD - E,), N, dtype=jnp.int32)
    srcp = jnp.concatenate([src, pad]).reshape(NROW, CH)
    dstp = jnp.concatenate([dst, pad]).reshape(NROW, CH)

    x_pad = jnp.pad(x, ((0, N_PAD - N), (0, 0)))
    ones_16 = jnp.ones((N_PAD, 16), jnp.float32)
    zeros_16 = jnp.zeros((N_PAD, 16), jnp.float32)

    degp = _deg_kernel(dstp, ones_16, zeros_16)
    y1, dis = _tc_b(x_pad, degp, W1)
    p1 = _edge_kernel(y1, srcp, dstp)
    y2 = _tc_d(p1, dis, W2, b1.reshape(1, D))
    p2 = _edge_kernel(y2, srcp, dstp)
    out = _tc_f(p2, dis, b2.reshape(1, D))
    return out[:N]


# speedup vs baseline: 12.2506x; 1.0170x over previous
import functools

import jax
import jax.numpy as jnp
from jax import lax
from jax.experimental import pallas as pl
from jax.experimental.pallas import tpu as pltpu
from jax.experimental.pallas import tpu_sc as plsc

N = 10000
E = 320000
D = 128
DH = D // 2

NC = 2
NS = 16
NW = NC * NS

N_PAD = 10240
ROWS_PT = N_PAD // NS
CH = 128
NROW = 2560
E_PAD = NROW * CH
NCH_DEG = NROW // NW
NCH_EDGE = NROW // NS
GRP = 4
EGRP = 4

BLK = 1280
GRID = N_PAD // BLK

_mesh = plsc.VectorSubcoreMesh(core_axis_name="c", subcore_axis_name="s")


@functools.partial(
    pl.kernel,
    out_type=jax.ShapeDtypeStruct((NC, N_PAD, 16), jnp.float32),
    mesh=_mesh,
    scratch_types=[
        pltpu.VMEM((NCH_DEG, CH), jnp.int32),
        pltpu.VMEM((CH, 16), jnp.float32),
        pltpu.VMEM_SHARED((N_PAD, 16), jnp.float32),
    ],
)
def _deg_kernel(dst_hbm, ones_hbm, zeros_hbm, out_hbm, dst_v, ones_v, acc):
    c = lax.axis_index("c")
    t = lax.axis_index("s")
    w = t * NC + c
    sl = pl.ds(t * ROWS_PT, ROWS_PT)

    pltpu.sync_copy(dst_hbm.at[pl.ds(w * NCH_DEG, NCH_DEG)], dst_v)
    pltpu.sync_copy(ones_hbm.at[pl.ds(0, CH)], ones_v)

    @pl.when(c == 0)
    def _():
        pltpu.sync_copy(ones_hbm.at[sl], acc.at[sl])

    @pl.when(c != 0)
    def _():
        pltpu.sync_copy(zeros_hbm.at[sl], acc.at[sl])

    plsc.subcore_barrier()

    def body(i, carry):
        for k in range(GRP):
            pltpu.sync_copy(ones_v, acc.at[dst_v.at[i * GRP + k]], add=True)
        return carry

    lax.fori_loop(0, NCH_DEG // GRP, body, 0)

    plsc.subcore_barrier()
    pltpu.sync_copy(acc.at[sl], out_hbm.at[c, sl])


@functools.partial(
    pl.kernel,
    out_type=jax.ShapeDtypeStruct((NC, N_PAD, DH), jnp.float32),
    mesh=_mesh,
    scratch_types=[
        pltpu.VMEM((NCH_EDGE, CH), jnp.int32),
        pltpu.VMEM((NCH_EDGE, CH), jnp.int32),
        pltpu.VMEM((EGRP, CH, DH), jnp.float32),
        pltpu.VMEM_SHARED((N_PAD, DH), jnp.float32),
        pltpu.SemaphoreType.DMA,
        pltpu.SemaphoreType.DMA,
    ],
    compiler_params=pltpu.CompilerParams(use_tc_tiling_on_sc=False),
)
def _edge_kernel(y_hbm, src_hbm, dst_hbm, out_hbm, src_v, dst_v, rows_v, acc,
                 sem, sem_s):
    c = lax.axis_index("c")
    t = lax.axis_index("s")
    sl = pl.ds(t * ROWS_PT, ROWS_PT)
    y_c = y_hbm.at[c]

    pltpu.sync_copy(src_hbm.at[pl.ds(t * NCH_EDGE, NCH_EDGE)], src_v)
    pltpu.sync_copy(dst_hbm.at[pl.ds(t * NCH_EDGE, NCH_EDGE)], dst_v)

    pltpu.sync_copy(y_c.at[sl], acc.at[sl])
    plsc.subcore_barrier()

    def body(i, carry):
        base = i * EGRP
        gathers = [
            pltpu.async_copy(y_c.at[src_v.at[base + k]], rows_v.at[k], sem)
            for k in range(EGRP)
        ]
        for g in gathers:
            g.wait()
        scatters = [
            pltpu.async_copy(rows_v.at[k], acc.at[dst_v.at[base + k]], sem_s,
                             add=True)
            for k in range(EGRP)
        ]
        for s_ in scatters:
            s_.wait()
        return carry

    lax.fori_loop(0, NCH_EDGE // EGRP, body, 0)

    plsc.subcore_barrier()
    pltpu.sync_copy(acc.at[sl], out_hbm.at[c, sl])


def _tc_b_body(x_ref, degp_ref, w_ref, y_ref, dis_ref):
    deg = degp_ref[0, :, 0:1] + degp_ref[1, :, 0:1]
    disb = jnp.broadcast_to(lax.rsqrt(deg), (BLK, D))
    xw = jnp.dot(x_ref[...], w_ref[...], preferred_element_type=jnp.float32)
    y = xw * disb
    y_ref[0] = y[:, :DH]
    y_ref[1] = y[:, DH:]
    dis_ref[...] = disb


def _tc_d_body(p_ref, dis_ref, w_ref, b_ref, y2_ref):
    dis = dis_ref[...]
    pre = jnp.concatenate([p_ref[0], p_ref[1]], axis=1)
    h = jnp.maximum(pre * dis + b_ref[...], 0.0)
    y2 = jnp.dot(h, w_ref[...], preferred_element_type=jnp.float32) * dis
    y2_ref[0] = y2[:, :DH]
    y2_ref[1] = y2[:, DH:]


def _tc_f_body(p_ref, dis_ref, b_ref, o_ref):
    pre = jnp.concatenate([p_ref[0], p_ref[1]], axis=1)
    o_ref[...] = pre * dis_ref[...] + b_ref[...]


_row_spec = pl.BlockSpec((BLK, D), lambda i: (i, 0))
_half_spec = pl.BlockSpec((NC, BLK, DH), lambda i: (0, i, 0))
_full_spec = pl.BlockSpec((D, D), lambda i: (0, 0))
_bias_spec = pl.BlockSpec((1, D), lambda i: (0, 0))
_y3 = jax.ShapeDtypeStruct((NC, N_PAD, DH), jnp.float32)

_tc_b = pl.pallas_call(
    _tc_b_body,
    grid=(GRID,),
    in_specs=[_row_spec, pl.BlockSpec((NC, BLK, 16), lambda i: (0, i, 0)),
              _full_spec],
    out_specs=[_half_spec, _row_spec],
    out_shape=[_y3, jax.ShapeDtypeStruct((N_PAD, D), jnp.float32)],
)

_tc_d = pl.pallas_call(
    _tc_d_body,
    grid=(GRID,),
    in_specs=[_half_spec, _row_spec, _full_spec, _bias_spec],
    out_specs=_half_spec,
    out_shape=_y3,
)

_tc_f = pl.pallas_call(
    _tc_f_body,
    grid=(GRID,),
    in_specs=[_half_spec, _row_spec, _bias_spec],
    out_specs=_row_spec,
    out_shape=jax.ShapeDtypeStruct((N_PAD, D), jnp.float32),
)


def kernel(x, edge_index, W1, b1, W2, b2):
    src = edge_index[0].astype(jnp.int32)
    dst = edge_index[1].astype(jnp.int32)

    pad = jnp.full((E_PAD - E,), N, dtype=jnp.int32)
    srcp = jnp.concatenate([src, pad]).reshape(NROW, CH)
    dstp = jnp.concatenate([dst, pad]).reshape(NROW, CH)

    x_pad = jnp.pad(x, ((0, N_PAD - N), (0, 0)))
    ones_16 = jnp.ones((N_PAD, 16), jnp.float32)
    zeros_16 = jnp.zeros((N_PAD, 16), jnp.float32)

    degp = _deg_kernel(dstp, ones_16, zeros_16)
    y1, dis = _tc_b(x_pad, degp, W1)
    p1 = _edge_kernel(y1, srcp, dstp)
    y2 = _tc_d(p1, dis, W2, b1.reshape(1, D))
    p2 = _edge_kernel(y2, srcp, dstp)
    out = _tc_f(p2, dis, b2.reshape(1, D))
    return out[:N]
